# baseline (speedup 1.0000x reference)
import functools

import jax
import jax.numpy as jnp
from jax import lax
from jax.experimental import pallas as pl
from jax.experimental.pallas import tpu as pltpu
from jax.experimental.pallas import tpu_sc as plsc

NC = 2
NS = 16
NW = NC * NS
SUB = 128
KSTREAM = 16


def _edge_pass(n_acc, n_rows, s_blocks, xp_hbm, col_hbm, row_hbm, zeros_hbm,
               out_hbm, table, acc, colb, rowb, data, sem_g, sem_s):
    c = lax.axis_index("c")
    s = lax.axis_index("s")
    zrows = n_acc // NS
    z0 = s * zrows
    t0 = jnp.minimum(s * zrows, n_rows - zrows)
    pltpu.sync_copy(xp_hbm.at[pl.ds(t0, zrows)], table.at[pl.ds(t0, zrows)])
    pltpu.sync_copy(zeros_hbm.at[pl.ds(z0, zrows)], acc.at[pl.ds(z0, zrows)])
    plsc.subcore_barrier()
    b0, b1 = s_blocks
    cnt = jnp.where(c == 0, b0, b1)
    base = (jnp.where(c == 0, s * b0, NS * b0 + s * b1)) * KSTREAM

    def block(b, carry):
        rb = base + b * KSTREAM
        pltpu.sync_copy(col_hbm.at[pl.ds(rb, KSTREAM)], colb)
        pltpu.sync_copy(row_hbm.at[pl.ds(rb, KSTREAM)], rowb)
        for j in range(KSTREAM):
            pltpu.async_copy(table.at[colb.at[j]], data.at[j], sem_g)
        for j in range(KSTREAM):
            pltpu.make_async_copy(table.at[colb.at[j]], data.at[j], sem_g).wait()
        for j in range(KSTREAM):
            pltpu.async_copy(data.at[j], acc.at[rowb.at[j]], sem_s, add=True)
        for j in range(KSTREAM):
            pltpu.make_async_copy(data.at[j], acc.at[rowb.at[j]], sem_s).wait()
        return carry

    lax.fori_loop(0, cnt, block, 0)
    plsc.subcore_barrier()
    pltpu.sync_copy(acc.at[pl.ds(z0, zrows)], out_hbm.at[c, pl.ds(z0, zrows)])


def _gather_pass(bsz, tabs_and_outs, u_hbm, i_hbm, ub, ib, dbuf, sem):
    c = lax.axis_index("c")
    s = lax.axis_index("s")
    w = c * NS + s
    rows_per_w = (bsz // SUB) // NW
    pltpu.sync_copy(u_hbm.at[pl.ds(w * rows_per_w, rows_per_w)], ub)
    pltpu.sync_copy(i_hbm.at[pl.ds(w * rows_per_w, rows_per_w)], ib)
    for idxb, which in ((ub, 0), (ib, 1)):
        for tab, outs in tabs_and_outs:
            out = outs[which]
            for j in range(rows_per_w):
                pltpu.async_copy(tab.at[idxb.at[j]], dbuf.at[j], sem)
            for j in range(rows_per_w):
                pltpu.make_async_copy(tab.at[idxb.at[j]], dbuf.at[j], sem).wait()
            for j in range(rows_per_w):
                pltpu.sync_copy(
                    dbuf.at[j],
                    out.at[pl.ds(w * rows_per_w * SUB + j * SUB, SUB)])


def kernel(u, i, adj_row, adj_col, adj_vals, embeddings, Wg1, bg1, Wg2, bg2,
           Wg3, bg3, Wu1, bu1, Wu2, bu2, Wi1, bi1, Wi2, bi2, Wc1, bc1, Wc2,
           bc2, Wo, bo):
    n, emb = embeddings.shape
    e = adj_row.shape[0]
    bsz = u.shape[0]
    e_rand = e - n

    block_edges = KSTREAM * SUB
    sb = -(-e // (NW * block_edges))
    b0 = (2 * sb * 44 + 50) // 102
    b1 = 2 * sb - b0
    e_pad = NS * (b0 + b1) * block_edges
    n_acc = -(-n // (NS * 8)) * (NS * 8)
    if n_acc == n:
        n_acc += NS * 8

    pad = e_pad - e
    row_p = jnp.concatenate(
        [adj_row.astype(jnp.int32), jnp.full((pad,), n, jnp.int32)]
    ).reshape(e_pad // SUB, SUB)
    col_p = jnp.concatenate(
        [adj_col.astype(jnp.int32), jnp.zeros((pad,), jnp.int32)]
    ).reshape(e_pad // SUB, SUB)
    zeros = jnp.zeros((n_acc, emb), jnp.float32)
    u2d = u.astype(jnp.int32).reshape(bsz // SUB, SUB)
    i2d = i.astype(jnp.int32).reshape(bsz // SUB, SUB)

    npr = 128 // emb
    nrow = n_acc // npr
    fpad = (n_acc - n) * emb
    v8 = jnp.pad(jnp.repeat(adj_vals[e_rand:], emb),
                 (0, fpad)).reshape(nrow, 128)
    embf = jnp.pad(embeddings.reshape(n * emb), (0, fpad)).reshape(nrow, 128)
    kron = [jnp.kron(jnp.eye(npr, dtype=jnp.float32), W)
            for W in (Wg1, Wg2, Wg3)]
    btile = [jnp.tile(b, npr).reshape(1, 128) for b in (bg1, bg2, bg3)]

    bn = nrow // 2
    nblk = nrow // bn

    def _pre(v8_ref, emb_ref, d8_ref, xp0_ref):
        d = jnp.sqrt(v8_ref[...])
        d8_ref[...] = d
        xp0_ref[...] = emb_ref[...] * d

    d8f, xp0f = pl.pallas_call(
        _pre,
        grid=(nblk,),
        in_specs=[
            pl.BlockSpec((bn, 128), lambda j: (j, 0)),
            pl.BlockSpec((bn, 128), lambda j: (j, 0)),
        ],
        out_specs=[
            pl.BlockSpec((bn, 128), lambda j: (j, 0)),
            pl.BlockSpec((bn, 128), lambda j: (j, 0)),
        ],
        out_shape=[
            jax.ShapeDtypeStruct((nrow, 128), jnp.float32),
            jax.ShapeDtypeStruct((nrow, 128), jnp.float32),
        ],
    )(v8, embf)
    xp0 = xp0f.reshape(n_acc, emb)

    mesh = plsc.VectorSubcoreMesh(core_axis_name="c", subcore_axis_name="s")
    edge_call = functools.partial(
        pl.kernel,
        out_type=jax.ShapeDtypeStruct((NC, n_acc, emb), jnp.float32),
        mesh=mesh,
        compiler_params=pltpu.CompilerParams(use_tc_tiling_on_sc=False),
        scratch_types=[
            pltpu.VMEM_SHARED((n_acc, emb), jnp.float32),
            pltpu.VMEM_SHARED((n_acc, emb), jnp.float32),
            pltpu.VMEM((KSTREAM, SUB), jnp.int32),
            pltpu.VMEM((KSTREAM, SUB), jnp.int32),
            pltpu.VMEM((KSTREAM, SUB, emb), jnp.float32),
            pltpu.SemaphoreType.DMA,
            pltpu.SemaphoreType.DMA,
        ],
    )(functools.partial(_edge_pass, n_acc, n, (b0, b1)))

    def _comb(p_ref, d8_ref, k_ref, b_ref, xp_ref, h_ref):
        d = d8_ref[...]
        xd = (p_ref[0] + p_ref[1]) * d
        z = jnp.dot(xd, k_ref[...],
                    preferred_element_type=jnp.float32) + b_ref[...]
        hh = jnp.maximum(z, 0.0)
        h_ref[...] = hh
        xp_ref[...] = hh * d

    comb_call = pl.pallas_call(
        _comb,
        grid=(nblk,),
        in_specs=[
            pl.BlockSpec((NC, bn, 128), lambda j: (0, j, 0)),
            pl.BlockSpec((bn, 128), lambda j: (j, 0)),
            pl.BlockSpec((128, 128), lambda j: (0, 0)),
            pl.BlockSpec((1, 128), lambda j: (0, 0)),
        ],
        out_specs=[
            pl.BlockSpec((bn, 128), lambda j: (j, 0)),
            pl.BlockSpec((bn, 128), lambda j: (j, 0)),
        ],
        out_shape=[
            jax.ShapeDtypeStruct((nrow, 128), jnp.float32),
            jax.ShapeDtypeStruct((nrow, 128), jnp.float32),
        ],
    )

    xp, hs = xp0, [embeddings]
    for k3 in range(3):
        parts = edge_call(xp, col_p, row_p, zeros)
        parts3 = parts.reshape(NC, nrow, 128)
        xpf, hf = comb_call(parts3, d8f, kron[k3], btile[k3])
        xp = xpf.reshape(n_acc, emb)
        hs.append(hf.reshape(n_acc, emb))

    def _gath(h0, h1, h2, h3, u_hbm, i_hbm,
              u0, u1, u2, u3, v0, v1, v2, v3, ub, ib, dbuf, sem):
        _gather_pass(bsz,
                     [(h0, (u0, v0)), (h1, (u1, v1)),
                      (h2, (u2, v2)), (h3, (u3, v3))],
                     u_hbm, i_hbm, ub, ib, dbuf, sem)

    rows_per_w = (bsz // SUB) // NW
    feats = pl.kernel(
        _gath,
        out_type=[jax.ShapeDtypeStruct((bsz, emb), jnp.float32)] * 8,
        mesh=mesh,
        compiler_params=pltpu.CompilerParams(use_tc_tiling_on_sc=False),
        scratch_types=[
            pltpu.VMEM((rows_per_w, SUB), jnp.int32),
            pltpu.VMEM((rows_per_w, SUB), jnp.int32),
            pltpu.VMEM((rows_per_w, SUB, emb), jnp.float32),
            pltpu.SemaphoreType.DMA,
        ],
    )(hs[0], hs[1], hs[2], hs[3], u2d, i2d)

    bb = 2048
    def _mlp(u0, u1, u2, u3, v0, v1, v2, v3,
             wu1, bu1r, wu2, bu2r, wi1, bi1r, wi2, bi2r,
             wc1, bc1r, wc2, bc2r, wo, bor, out_ref):
        uf = jnp.concatenate([u0[...], u1[...], u2[...], u3[...]], axis=1)
        vf = jnp.concatenate([v0[...], v1[...], v2[...], v3[...]], axis=1)
        dot = lambda a, w, b_: jnp.dot(
            a, w[...], preferred_element_type=jnp.float32) + b_[...]
        uh = jnp.maximum(dot(uf, wu1, bu1r), 0.0)
        uh = jnp.maximum(dot(uh, wu2, bu2r), 0.0)
        vh = jnp.maximum(dot(vf, wi1, bi1r), 0.0)
        vh = jnp.maximum(dot(vh, wi2, bi2r), 0.0)
        z = jnp.concatenate([uh, vh], axis=1)
        z = jnp.maximum(dot(z, wc1, bc1r), 0.0)
        z = jnp.maximum(dot(z, wc2, bc2r), 0.0)
        out_ref[...] = jax.nn.sigmoid(dot(z, wo, bor))

    feat_spec = pl.BlockSpec((bb, emb), lambda j: (j, 0))
    full = lambda a: pl.BlockSpec(a.shape, lambda j: tuple(0 for _ in a.shape))
    wlist = [Wu1, bu1.reshape(1, -1), Wu2, bu2.reshape(1, -1),
             Wi1, bi1.reshape(1, -1), Wi2, bi2.reshape(1, -1),
             Wc1, bc1.reshape(1, -1), Wc2, bc2.reshape(1, -1),
             Wo, bo.reshape(1, -1)]
    out = pl.pallas_call(
        _mlp,
        grid=(bsz // bb,),
        in_specs=[feat_spec] * 8 + [full(a) for a in wlist],
        out_specs=pl.BlockSpec((bb, 1), lambda j: (j, 0)),
        out_shape=jax.ShapeDtypeStruct((bsz, 1), jnp.float32),
    )(*feats, *wlist)
    return out


# speedup vs baseline: 1.1403x; 1.1403x over previous
import functools

import jax
import jax.numpy as jnp
from jax import lax
from jax.experimental import pallas as pl
from jax.experimental.pallas import tpu as pltpu
from jax.experimental.pallas import tpu_sc as plsc

NC = 2
NS = 16
NW = NC * NS
SUB = 128
KSTREAM = 16


def _edge_pass(n_acc, n_rows, s_blocks, xp_hbm, col_hbm, row_hbm, zeros_hbm,
               out_hbm, table, acc, colb, rowb, data, sem_g, sem_s):
    c = lax.axis_index("c")
    s = lax.axis_index("s")
    zrows = n_acc // NS
    z0 = s * zrows
    t0 = jnp.minimum(s * zrows, n_rows - zrows)
    pltpu.sync_copy(xp_hbm.at[pl.ds(t0, zrows)], table.at[pl.ds(t0, zrows)])
    pltpu.sync_copy(zeros_hbm.at[pl.ds(z0, zrows)], acc.at[pl.ds(z0, zrows)])
    plsc.subcore_barrier()
    b0, b1 = s_blocks
    cnt = jnp.where(c == 0, b0, b1)
    base = (jnp.where(c == 0, s * b0, NS * b0 + s * b1)) * KSTREAM

    def block(b, carry):
        rb = base + b * KSTREAM
        pltpu.sync_copy(col_hbm.at[pl.ds(rb, KSTREAM)], colb)
        pltpu.sync_copy(row_hbm.at[pl.ds(rb, KSTREAM)], rowb)
        for j in range(KSTREAM):
            pltpu.async_copy(table.at[colb.at[j]], data.at[j], sem_g)
        for j in range(KSTREAM):
            pltpu.make_async_copy(table.at[colb.at[j]], data.at[j], sem_g).wait()
        for j in range(KSTREAM):
            pltpu.async_copy(data.at[j], acc.at[rowb.at[j]], sem_s, add=True)
        for j in range(KSTREAM):
            pltpu.make_async_copy(data.at[j], acc.at[rowb.at[j]], sem_s).wait()
        return carry

    lax.fori_loop(0, cnt, block, 0)
    plsc.subcore_barrier()
    pltpu.sync_copy(acc.at[pl.ds(z0, zrows)], out_hbm.at[c, pl.ds(z0, zrows)])


def _gather_pass(bsz, tabs_and_outs, u_hbm, i_hbm, ub, ib, dbuf, sem):
    c = lax.axis_index("c")
    s = lax.axis_index("s")
    w = c * NS + s
    rows_per_w = (bsz // SUB) // NW
    pltpu.sync_copy(u_hbm.at[pl.ds(w * rows_per_w, rows_per_w)], ub)
    pltpu.sync_copy(i_hbm.at[pl.ds(w * rows_per_w, rows_per_w)], ib)
    for idxb, which in ((ub, 0), (ib, 1)):
        for tab, outs in tabs_and_outs:
            out = outs[which]
            for j in range(rows_per_w):
                pltpu.async_copy(tab.at[idxb.at[j]], dbuf.at[j], sem)
            for j in range(rows_per_w):
                pltpu.make_async_copy(tab.at[idxb.at[j]], dbuf.at[j], sem).wait()
            for j in range(rows_per_w):
                pltpu.sync_copy(
                    dbuf.at[j],
                    out.at[pl.ds(w * rows_per_w * SUB + j * SUB, SUB)])


def kernel(u, i, adj_row, adj_col, adj_vals, embeddings, Wg1, bg1, Wg2, bg2,
           Wg3, bg3, Wu1, bu1, Wu2, bu2, Wi1, bi1, Wi2, bi2, Wc1, bc1, Wc2,
           bc2, Wo, bo):
    n, emb = embeddings.shape
    e = adj_row.shape[0]
    bsz = u.shape[0]
    e_rand = e - n

    block_edges = KSTREAM * SUB
    sb = -(-e // (NW * block_edges))
    b0 = (2 * sb * 58 + 50) // 102
    b1 = 2 * sb - b0
    e_pad = NS * (b0 + b1) * block_edges
    n_acc = -(-n // (NS * 8)) * (NS * 8)
    if n_acc == n:
        n_acc += NS * 8

    pad = e_pad - e
    row_p = jnp.concatenate(
        [adj_row.astype(jnp.int32), jnp.full((pad,), n, jnp.int32)]
    ).reshape(e_pad // SUB, SUB)
    col_p = jnp.concatenate(
        [adj_col.astype(jnp.int32), jnp.zeros((pad,), jnp.int32)]
    ).reshape(e_pad // SUB, SUB)
    zeros = jnp.zeros((n_acc, emb), jnp.float32)
    u2d = u.astype(jnp.int32).reshape(bsz // SUB, SUB)
    i2d = i.astype(jnp.int32).reshape(bsz // SUB, SUB)

    npr = 128 // emb
    nrow = n_acc // npr
    fpad = (n_acc - n) * emb
    v8 = jnp.pad(jnp.repeat(adj_vals[e_rand:], emb),
                 (0, fpad)).reshape(nrow, 128)
    embf = jnp.pad(embeddings.reshape(n * emb), (0, fpad)).reshape(nrow, 128)
    kron = [jnp.kron(jnp.eye(npr, dtype=jnp.float32), W)
            for W in (Wg1, Wg2, Wg3)]
    btile = [jnp.tile(b, npr).reshape(1, 128) for b in (bg1, bg2, bg3)]

    bn = nrow // 2
    nblk = nrow // bn

    def _pre(v8_ref, emb_ref, d8_ref, xp0_ref):
        d = jnp.sqrt(v8_ref[...])
        d8_ref[...] = d
        xp0_ref[...] = emb_ref[...] * d

    d8f, xp0f = pl.pallas_call(
        _pre,
        grid=(nblk,),
        in_specs=[
            pl.BlockSpec((bn, 128), lambda j: (j, 0)),
            pl.BlockSpec((bn, 128), lambda j: (j, 0)),
        ],
        out_specs=[
            pl.BlockSpec((bn, 128), lambda j: (j, 0)),
            pl.BlockSpec((bn, 128), lambda j: (j, 0)),
        ],
        out_shape=[
            jax.ShapeDtypeStruct((nrow, 128), jnp.float32),
            jax.ShapeDtypeStruct((nrow, 128), jnp.float32),
        ],
    )(v8, embf)
    xp0 = xp0f.reshape(n_acc, emb)

    mesh = plsc.VectorSubcoreMesh(core_axis_name="c", subcore_axis_name="s")
    edge_call = functools.partial(
        pl.kernel,
        out_type=jax.ShapeDtypeStruct((NC, n_acc, emb), jnp.float32),
        mesh=mesh,
        compiler_params=pltpu.CompilerParams(use_tc_tiling_on_sc=False),
        scratch_types=[
            pltpu.VMEM_SHARED((n_acc, emb), jnp.float32),
            pltpu.VMEM_SHARED((n_acc, emb), jnp.float32),
            pltpu.VMEM((KSTREAM, SUB), jnp.int32),
            pltpu.VMEM((KSTREAM, SUB), jnp.int32),
            pltpu.VMEM((KSTREAM, SUB, emb), jnp.float32),
            pltpu.SemaphoreType.DMA,
            pltpu.SemaphoreType.DMA,
        ],
    )(functools.partial(_edge_pass, n_acc, n, (b0, b1)))

    def _comb(p_ref, d8_ref, k_ref, b_ref, xp_ref, h_ref):
        d = d8_ref[...]
        xd = (p_ref[0] + p_ref[1]) * d
        z = jnp.dot(xd, k_ref[...],
                    preferred_element_type=jnp.float32) + b_ref[...]
        hh = jnp.maximum(z, 0.0)
        h_ref[...] = hh
        xp_ref[...] = hh * d

    comb_call = pl.pallas_call(
        _comb,
        grid=(nblk,),
        in_specs=[
            pl.BlockSpec((NC, bn, 128), lambda j: (0, j, 0)),
            pl.BlockSpec((bn, 128), lambda j: (j, 0)),
            pl.BlockSpec((128, 128), lambda j: (0, 0)),
            pl.BlockSpec((1, 128), lambda j: (0, 0)),
        ],
        out_specs=[
            pl.BlockSpec((bn, 128), lambda j: (j, 0)),
            pl.BlockSpec((bn, 128), lambda j: (j, 0)),
        ],
        out_shape=[
            jax.ShapeDtypeStruct((nrow, 128), jnp.float32),
            jax.ShapeDtypeStruct((nrow, 128), jnp.float32),
        ],
    )

    xp, hs = xp0, [embeddings]
    for k3 in range(3):
        parts = edge_call(xp, col_p, row_p, zeros)
        parts3 = parts.reshape(NC, nrow, 128)
        xpf, hf = comb_call(parts3, d8f, kron[k3], btile[k3])
        xp = xpf.reshape(n_acc, emb)
        hs.append(hf.reshape(n_acc, emb))

    def _gath(h0, h1, h2, h3, u_hbm, i_hbm,
              u0, u1, u2, u3, v0, v1, v2, v3, ub, ib, dbuf, sem):
        _gather_pass(bsz,
                     [(h0, (u0, v0)), (h1, (u1, v1)),
                      (h2, (u2, v2)), (h3, (u3, v3))],
                     u_hbm, i_hbm, ub, ib, dbuf, sem)

    rows_per_w = (bsz // SUB) // NW
    feats = pl.kernel(
        _gath,
        out_type=[jax.ShapeDtypeStruct((bsz, emb), jnp.float32)] * 8,
        mesh=mesh,
        compiler_params=pltpu.CompilerParams(use_tc_tiling_on_sc=False),
        scratch_types=[
            pltpu.VMEM((rows_per_w, SUB), jnp.int32),
            pltpu.VMEM((rows_per_w, SUB), jnp.int32),
            pltpu.VMEM((rows_per_w, SUB, emb), jnp.float32),
            pltpu.SemaphoreType.DMA,
        ],
    )(hs[0], hs[1], hs[2], hs[3], u2d, i2d)

    bb = 2048
    def _mlp(u0, u1, u2, u3, v0, v1, v2, v3,
             wu1, bu1r, wu2, bu2r, wi1, bi1r, wi2, bi2r,
             wc1, bc1r, wc2, bc2r, wo, bor, out_ref):
        uf = jnp.concatenate([u0[...], u1[...], u2[...], u3[...]], axis=1)
        vf = jnp.concatenate([v0[...], v1[...], v2[...], v3[...]], axis=1)
        dot = lambda a, w, b_: jnp.dot(
            a, w[...], preferred_element_type=jnp.float32) + b_[...]
        uh = jnp.maximum(dot(uf, wu1, bu1r), 0.0)
        uh = jnp.maximum(dot(uh, wu2, bu2r), 0.0)
        vh = jnp.maximum(dot(vf, wi1, bi1r), 0.0)
        vh = jnp.maximum(dot(vh, wi2, bi2r), 0.0)
        z = jnp.concatenate([uh, vh], axis=1)
        z = jnp.maximum(dot(z, wc1, bc1r), 0.0)
        z = jnp.maximum(dot(z, wc2, bc2r), 0.0)
        out_ref[...] = jax.nn.sigmoid(dot(z, wo, bor))

    feat_spec = pl.BlockSpec((bb, emb), lambda j: (j, 0))
    full = lambda a: pl.BlockSpec(a.shape, lambda j: tuple(0 for _ in a.shape))
    wlist = [Wu1, bu1.reshape(1, -1), Wu2, bu2.reshape(1, -1),
             Wi1, bi1.reshape(1, -1), Wi2, bi2.reshape(1, -1),
             Wc1, bc1.reshape(1, -1), Wc2, bc2.reshape(1, -1),
             Wo, bo.reshape(1, -1)]
    out = pl.pallas_call(
        _mlp,
        grid=(bsz // bb,),
        in_specs=[feat_spec] * 8 + [full(a) for a in wlist],
        out_specs=pl.BlockSpec((bb, 1), lambda j: (j, 0)),
        out_shape=jax.ShapeDtypeStruct((bsz, 1), jnp.float32),
    )(*feats, *wlist)
    return out
